# baseline (speedup 1.0000x reference)
import functools

import jax
import jax.numpy as jnp
from jax import lax
from jax.experimental import pallas as pl
from jax.experimental.pallas import tpu as pltpu
from jax.experimental.pallas import tpu_sc as plsc

N = 10000
E = 320000
D = 128
HD = D // 2
R = 8
NBASES = 30
RN = R * N

NC = 2
NS = 16
NW = NC * NS
L = 16

K = 128
CHUNKS = E // K

_GDN = lax.GatherDimensionNumbers(
    offset_dims=(), collapsed_slice_dims=(0,), start_index_map=(0,))


def _bcast_lane(vec, j):
  idx = jnp.full((L, 1), j, jnp.int32)
  return lax.gather(vec, idx, _GDN, slice_sizes=(1,),
                    mode=lax.GatherScatterMode.PROMISE_IN_BOUNDS)


def _num_chunks_for(wid):
  return (CHUNKS - wid + NW - 1) // NW


def _t0_body(comp_ref, basis_ref, out_ref):
  out_ref[...] = jnp.dot(comp_ref[...], basis_ref[...],
                         preferred_element_type=jnp.float32)


def _t0(comp, basis2d):
  g = basis2d.shape[1] // 2048
  return pl.pallas_call(
      _t0_body,
      grid=(g,),
      in_specs=[
          pl.BlockSpec((R, NBASES), lambda i: (0, 0)),
          pl.BlockSpec((NBASES, 2048), lambda i: (0, i)),
      ],
      out_specs=pl.BlockSpec((R, 2048), lambda i: (0, i)),
      out_shape=jax.ShapeDtypeStruct((R, basis2d.shape[1]), jnp.float32),
  )(comp, basis2d)


def _t1_body(x_ref, w_ref, b_ref, outa_ref, outb_ref):
  o = jnp.dot(x_ref[...], w_ref[0], preferred_element_type=jnp.float32)
  bias = jnp.where(pl.program_id(0) == R, b_ref[...], 0.0)
  o = o + bias
  outa_ref[0] = o[:, :HD]
  outb_ref[0] = o[:, HD:]


def _t1(x, w_all, root_b2d):
  bn = 400
  g = N // bn
  half = jax.ShapeDtypeStruct((R + 1, N, HD), jnp.float32)
  hspec = pl.BlockSpec((1, bn, HD), lambda r, i: (r, i, 0))
  return pl.pallas_call(
      _t1_body,
      grid=(R + 1, g),
      in_specs=[
          pl.BlockSpec((bn, D), lambda r, i: (i, 0)),
          pl.BlockSpec((1, D, D), lambda r, i: (r, 0, 0)),
          pl.BlockSpec((1, D), lambda r, i: (0, 0)),
      ],
      out_specs=[hspec, hspec],
      out_shape=[half, half],
  )(x, w_all, root_b2d)


def _t2_body(h_ref, out_ref):
  s = jnp.sum(h_ref[...], axis=0, keepdims=True)
  out_ref[...] = 1.0 / jnp.maximum(s, 1.0)


def _t2(hists):
  bl = 3200
  g = RN // bl
  return pl.pallas_call(
      _t2_body,
      grid=(g,),
      in_specs=[pl.BlockSpec((NW, bl), lambda i: (0, i))],
      out_specs=pl.BlockSpec((1, bl), lambda i: (0, i)),
      out_shape=jax.ShapeDtypeStruct((1, RN), jnp.float32),
  )(hists)


def _t3_body(xa_ref, a0_ref, a1_ref, xb_ref, b0_ref, b1_ref,
             oa_ref, ob_ref):
  oa_ref[...] = xa_ref[...] + a0_ref[...] + a1_ref[...]
  ob_ref[...] = xb_ref[...] + b0_ref[...] + b1_ref[...]


def _t3(xa, a0, a1, xb, b0, b1):
  bn = 400
  g = N // bn
  spec = pl.BlockSpec((bn, HD), lambda i: (i, 0))
  half = jax.ShapeDtypeStruct((N, HD), jnp.float32)
  return pl.pallas_call(
      _t3_body,
      grid=(g,),
      in_specs=[spec] * 6,
      out_specs=[spec, spec],
      out_shape=[half, half],
  )(xa, a0, a1, xb, b0, b1)


def _t4_body(na0_ref, na1_ref, nb0_ref, nb1_ref, o1a_ref, o1b_ref,
             wra_ref, wrb_ref, wwa_ref, wwb_ref, b_ref, out_ref):
  na = na0_ref[...] + na1_ref[...]
  nb = nb0_ref[...] + nb1_ref[...]
  f32 = jnp.float32
  out_ref[...] = (
      jnp.dot(na, wra_ref[...], preferred_element_type=f32)
      + jnp.dot(nb, wrb_ref[...], preferred_element_type=f32)
      + jnp.dot(o1a_ref[...], wwa_ref[...], preferred_element_type=f32)
      + jnp.dot(o1b_ref[...], wwb_ref[...], preferred_element_type=f32)
      + b_ref[...])


def _t4(na0, na1, nb0, nb1, o1a, o1b, w_rel, w_root, b2d):
  bn = 400
  g = N // bn
  nspec = pl.BlockSpec((bn, HD), lambda i: (i, 0))
  wspec = pl.BlockSpec((HD, D), lambda i: (0, 0))
  return pl.pallas_call(
      _t4_body,
      grid=(g,),
      in_specs=[nspec, nspec, nspec, nspec, nspec, nspec,
                wspec, wspec, wspec, wspec,
                pl.BlockSpec((1, D), lambda i: (0, 0))],
      out_specs=pl.BlockSpec((bn, D), lambda i: (i, 0)),
      out_shape=jax.ShapeDtypeStruct((N, D), jnp.float32),
  )(na0, na1, nb0, nb1, o1a, o1b,
    w_rel[:HD], w_rel[HD:], w_root[:HD], w_root[HD:], b2d)


_MESH = plsc.VectorSubcoreMesh(
    core_axis_name="c", subcore_axis_name="s", num_cores=NC, num_subcores=NS)
_SC_PARAMS = pltpu.CompilerParams(
    needs_layout_passes=False, use_tc_tiling_on_sc=False)


def _a1_body(idx3_hbm, out_hbm, hist_v, ibuf, isem):
  wid = lax.axis_index("s") * NC + lax.axis_index("c")
  nk = _num_chunks_for(wid)
  lane = lax.iota(jnp.int32, L)
  zeros = jnp.zeros((L,), jnp.float32)
  ones = jnp.ones((L,), jnp.float32)

  def zero_body(i, _):
    hist_v[pl.ds(i * L, L)] = zeros
    return 0
  lax.fori_loop(0, RN // L, zero_body, 0)

  def fetch(k, b):
    @pl.when(k < nk)
    def _():
      c = wid + k * NW
      pltpu.async_copy(idx3_hbm.at[c].at[pl.ds(1, 2)], ibuf.at[b],
                       isem.at[b])

  def process(k, b):
    @pl.when(k < nk)
    def _():
      pltpu.make_async_copy(idx3_hbm.at[wid].at[pl.ds(1, 2)], ibuf.at[b],
                            isem.at[b]).wait()

      def blk(i, _):
        dv = ibuf[b, 0, pl.ds(i * L, L)]
        tv = ibuf[b, 1, pl.ds(i * L, L)]
        civ = tv * N + dv
        for j in range(L):
          plsc.addupdate_scatter(hist_v, [civ], ones, mask=(lane == j))
        return 0
      lax.fori_loop(0, K // L, blk, 0)

  fetch(jnp.int32(0), 0)
  fetch(jnp.int32(1), 1)

  def body(kk, _):
    k = kk * 2
    process(k, 0)
    fetch(k + 2, 0)
    process(k + 1, 1)
    fetch(k + 3, 1)
    return 0
  lax.fori_loop(0, (nk + 1) // 2, body, 0)
  pltpu.sync_copy(hist_v, out_hbm.at[wid])


@functools.partial(
    pl.kernel,
    out_type=jax.ShapeDtypeStruct((NW, RN), jnp.float32),
    mesh=_MESH,
    scratch_types=[
        pltpu.VMEM((RN,), jnp.float32),
        pltpu.VMEM((2, 2, K), jnp.int32),
        pltpu.SemaphoreType.DMA((2,)),
    ],
    compiler_params=_SC_PARAMS,
)
def _sc_a1(idx3_hbm, out_hbm, hist_v, ibuf, isem):
  _a1_body(idx3_hbm, out_hbm, hist_v, ibuf, isem)


def _sub_blocks(sid):
  start = sid * 624
  nblk = jnp.where(sid == NS - 1, 40, 39)
  return start, nblk


def _zero_spmem_slice(agg_sh, rows_v, sid):
  def zr(i, _):
    for h in range(HD // L):
      rows_v[i, pl.ds(h * L, L)] = jnp.zeros((L,), jnp.float32)
    return 0
  lax.fori_loop(0, 16, zr, 0)
  start, nblk = _sub_blocks(sid)

  def cp(k, _):
    off = pl.multiple_of(start + k * 16, 8)
    pltpu.sync_copy(rows_v.at[pl.ds(0, 16)], agg_sh.at[pl.ds(off, 16)])
    return 0
  lax.fori_loop(0, nblk, cp, 0)


def _write_out_slice(agg_sh, out_hbm, sid):
  start, nblk = _sub_blocks(sid)

  def cp(k, _):
    off = pl.multiple_of(start + k * 16, 8)
    pltpu.sync_copy(agg_sh.at[pl.ds(off, 16)], out_hbm.at[pl.ds(off, 16)])
    return 0
  lax.fori_loop(0, nblk, cp, 0)


def _agg_pipeline(table_a, table_b, idx3_hbm, wt16_hbm, out_hbm,
                  agg_a, agg_b, ra, rb, ibuf, wbuf, gidx, widx, dstb,
                  isem, wsem, gsa, gsb, ssa, ssb, scaled):
  cid = lax.axis_index("c")
  sid = lax.axis_index("s")
  wid = sid * NC + cid
  nk = _num_chunks_for(wid)

  _zero_spmem_slice(agg_a, ra.at[0], sid)
  _zero_spmem_slice(agg_b, rb.at[0], sid)
  plsc.subcore_barrier()

  def fetch(k, b):
    @pl.when(k < nk)
    def _():
      c = wid + k * NW
      pltpu.async_copy(idx3_hbm.at[c], ibuf.at[b], isem.at[b])

  def gather(k, b):
    @pl.when(k < nk)
    def _():
      pltpu.make_async_copy(
          idx3_hbm.at[wid], ibuf.at[b], isem.at[b]).wait()
      if scaled:
        def gix(i, _):
          sv = ibuf[b, 0, pl.ds(i * L, L)]
          dv = ibuf[b, 1, pl.ds(i * L, L)]
          tv = ibuf[b, 2, pl.ds(i * L, L)]
          tvn = tv * N
          gidx[b, pl.ds(i * L, L)] = tvn + sv
          widx[b, pl.ds(i * L, L)] = tvn + dv
          return 0
        lax.fori_loop(0, K // L, gix, 0)
        src_idx = gidx.at[b]
        pltpu.async_copy(wt16_hbm.at[widx.at[b]], wbuf.at[b], wsem.at[b])
      else:
        src_idx = ibuf.at[b].at[0]

      @pl.when(k >= 2)
      def _():
        pltpu.make_async_copy(
            ra.at[b], agg_a.at[dstb.at[b]], ssa.at[b]).wait()
        pltpu.make_async_copy(
            rb.at[b], agg_b.at[dstb.at[b]], ssb.at[b]).wait()
      pltpu.async_copy(table_a.at[src_idx], ra.at[b], gsa.at[b])
      pltpu.async_copy(table_b.at[src_idx], rb.at[b], gsb.at[b])

  def process(k, b):
    @pl.when(k < nk)
    def _():
      if scaled:
        src_idx = gidx.at[b]
        pltpu.make_async_copy(
            wt16_hbm.at[widx.at[b]], wbuf.at[b], wsem.at[b]).wait()
      else:
        src_idx = ibuf.at[b].at[0]
      pltpu.make_async_copy(table_a.at[src_idx], ra.at[b], gsa.at[b]).wait()
      pltpu.make_async_copy(table_b.at[src_idx], rb.at[b], gsb.at[b]).wait()

      def blk(i, _):
        dstb[b, pl.ds(i * L, L)] = ibuf[b, 1, pl.ds(i * L, L)]
        if scaled:
          for j in range(L):
            e = i * L + j
            w = wbuf[b, e]
            for h in range(HD // L):
              ra[b, e, pl.ds(h * L, L)] = ra[b, e, pl.ds(h * L, L)] * w
              rb[b, e, pl.ds(h * L, L)] = rb[b, e, pl.ds(h * L, L)] * w
        return 0
      lax.fori_loop(0, K // L, blk, 0)

      pltpu.async_copy(ra.at[b], agg_a.at[dstb.at[b]], ssa.at[b], add=True)
      pltpu.async_copy(rb.at[b], agg_b.at[dstb.at[b]], ssb.at[b], add=True)

  fetch(jnp.int32(0), 0)
  fetch(jnp.int32(1), 1)
  gather(jnp.int32(0), 0)

  def body(kk, _):
    k = kk * 2
    process(k, 0)
    fetch(k + 2, 0)
    gather(k + 1, 1)
    process(k + 1, 1)
    fetch(k + 3, 1)
    gather(k + 2, 0)
    return 0
  lax.fori_loop(0, (nk + 1) // 2, body, 0)

  @pl.when(nk >= 2)
  def _():
    for b in range(2):
      pltpu.make_async_copy(
          ra.at[b], agg_a.at[dstb.at[b]], ssa.at[b]).wait()
      pltpu.make_async_copy(
          rb.at[b], agg_b.at[dstb.at[b]], ssb.at[b]).wait()

  @pl.when(nk == 1)
  def _():
    pltpu.make_async_copy(
        ra.at[0], agg_a.at[dstb.at[0]], ssa.at[0]).wait()
    pltpu.make_async_copy(
        rb.at[0], agg_b.at[dstb.at[0]], ssb.at[0]).wait()

  plsc.subcore_barrier()
  _write_out_slice(agg_a, out_hbm.at[0].at[cid], sid)
  _write_out_slice(agg_b, out_hbm.at[1].at[cid], sid)


_AGG_SCRATCH = [
    pltpu.VMEM_SHARED((N, HD), jnp.float32),
    pltpu.VMEM_SHARED((N, HD), jnp.float32),
    pltpu.VMEM((2, K, HD), jnp.float32),
    pltpu.VMEM((2, K, HD), jnp.float32),
    pltpu.VMEM((2, 3, K), jnp.int32),
    pltpu.VMEM((2, K, L), jnp.float32),
    pltpu.VMEM((2, K), jnp.int32),
    pltpu.VMEM((2, K), jnp.int32),
    pltpu.VMEM((2, K), jnp.int32),
    pltpu.SemaphoreType.DMA((2,)),
    pltpu.SemaphoreType.DMA((2,)),
    pltpu.SemaphoreType.DMA((2,)),
    pltpu.SemaphoreType.DMA((2,)),
    pltpu.SemaphoreType.DMA((2,)),
    pltpu.SemaphoreType.DMA((2,)),
]


@functools.partial(
    pl.kernel,
    out_type=jax.ShapeDtypeStruct((2, NC, N, HD), jnp.float32),
    mesh=_MESH,
    scratch_types=_AGG_SCRATCH,
    compiler_params=_SC_PARAMS,
)
def _sc_a2(ta, tb, idx3, wt16, out, agg_a, agg_b, ra, rb, ibuf, wbuf,
           gidx, widx, dstb, isem, wsem, gsa, gsb, ssa, ssb):
  _agg_pipeline(ta, tb, idx3, wt16, out, agg_a, agg_b, ra, rb, ibuf, wbuf,
                gidx, widx, dstb, isem, wsem, gsa, gsb, ssa, ssb,
                scaled=True)


@functools.partial(
    pl.kernel,
    out_type=jax.ShapeDtypeStruct((2, NC, N, HD), jnp.float32),
    mesh=_MESH,
    scratch_types=_AGG_SCRATCH,
    compiler_params=_SC_PARAMS,
)
def _sc_b(ta, tb, idx3, wt16, out, agg_a, agg_b, ra, rb, ibuf, wbuf,
          gidx, widx, dstb, isem, wsem, gsa, gsb, ssa, ssb):
  _agg_pipeline(ta, tb, idx3, wt16, out, agg_a, agg_b, ra, rb, ibuf, wbuf,
                gidx, widx, dstb, isem, wsem, gsa, gsb, ssa, ssb,
                scaled=False)


def kernel(node_features, edge_index, edge_norm, edge_type, basis, comp,
           root_w, root_b, gc_w_rel, gc_w_root, gc_b):
  del edge_norm
  src = edge_index[0].astype(jnp.int32)
  dst = edge_index[1].astype(jnp.int32)
  et = edge_type.astype(jnp.int32)
  idx3 = jnp.stack([src, dst, et]).reshape(3, CHUNKS, K).transpose(1, 0, 2)

  w8 = _t0(comp, basis.reshape(NBASES, D * D))
  w_all = jnp.concatenate(
      [w8.reshape(R, D, D), root_w[None]], axis=0)

  xca, xcb = _t1(node_features, w_all, root_b.reshape(1, D))
  tbl_a = xca.reshape((R + 1) * N, HD)
  tbl_b = xcb.reshape((R + 1) * N, HD)

  hists = _sc_a1(idx3)
  wtab = _t2(hists).reshape(RN)
  wt16 = jnp.broadcast_to(wtab[:, None], (RN, L))

  agg = _sc_a2(tbl_a, tbl_b, idx3, wt16)
  o1a, o1b = _t3(xca[R], agg[0, 0], agg[0, 1],
                 xcb[R], agg[1, 0], agg[1, 1])

  neigh = _sc_b(o1a, o1b, idx3, wt16)
  out2 = _t4(neigh[0, 0], neigh[0, 1], neigh[1, 0], neigh[1, 1],
             o1a, o1b, gc_w_rel, gc_w_root, gc_b.reshape(1, D))
  return out2


# speedup vs baseline: 1.0365x; 1.0365x over previous
import functools

import jax
import jax.numpy as jnp
from jax import lax
from jax.experimental import pallas as pl
from jax.experimental.pallas import tpu as pltpu
from jax.experimental.pallas import tpu_sc as plsc

N = 10000
E = 320000
D = 128
HD = D // 2
R = 8
NBASES = 30
RN = R * N

NC = 2
NS = 16
NW = NC * NS
L = 16

K = 128
CHUNKS = E // K

_GDN = lax.GatherDimensionNumbers(
    offset_dims=(), collapsed_slice_dims=(0,), start_index_map=(0,))


def _bcast_lane(vec, j):
  idx = jnp.full((L, 1), j, jnp.int32)
  return lax.gather(vec, idx, _GDN, slice_sizes=(1,),
                    mode=lax.GatherScatterMode.PROMISE_IN_BOUNDS)


def _num_chunks_for(wid):
  return (CHUNKS - wid + NW - 1) // NW


def _t0_body(comp_ref, basis_ref, out_ref):
  out_ref[...] = jnp.dot(comp_ref[...], basis_ref[...],
                         preferred_element_type=jnp.float32)


def _t0(comp, basis2d):
  g = basis2d.shape[1] // 2048
  return pl.pallas_call(
      _t0_body,
      grid=(g,),
      in_specs=[
          pl.BlockSpec((R, NBASES), lambda i: (0, 0)),
          pl.BlockSpec((NBASES, 2048), lambda i: (0, i)),
      ],
      out_specs=pl.BlockSpec((R, 2048), lambda i: (0, i)),
      out_shape=jax.ShapeDtypeStruct((R, basis2d.shape[1]), jnp.float32),
  )(comp, basis2d)


def _t1_body(x_ref, w_ref, b_ref, outa_ref, outb_ref):
  o = jnp.dot(x_ref[...], w_ref[0], preferred_element_type=jnp.float32)
  bias = jnp.where(pl.program_id(1) == R, b_ref[...], 0.0)
  o = o + bias
  outa_ref[...] = o[:, :HD]
  outb_ref[...] = o[:, HD:]


def _t1(x, w_all, root_b2d):
  bn = 400
  g = N // bn
  half = jax.ShapeDtypeStruct(((R + 1) * N, HD), jnp.float32)
  hspec = pl.BlockSpec((bn, HD), lambda i, r: (r * g + i, 0))
  return pl.pallas_call(
      _t1_body,
      grid=(g, R + 1),
      in_specs=[
          pl.BlockSpec((bn, D), lambda i, r: (i, 0)),
          pl.BlockSpec((1, D, D), lambda i, r: (r, 0, 0)),
          pl.BlockSpec((1, D), lambda i, r: (0, 0)),
      ],
      out_specs=[hspec, hspec],
      out_shape=[half, half],
  )(x, w_all, root_b2d)


def _t2_body(h_ref, out_ref):
  s = jnp.sum(h_ref[...], axis=0, keepdims=True)
  w = 1.0 / jnp.maximum(s, 1.0)
  out_ref[...] = jnp.broadcast_to(w.reshape(-1, 1), out_ref.shape)


def _t2(hists):
  bl = 3200
  g = RN // bl
  return pl.pallas_call(
      _t2_body,
      grid=(g,),
      in_specs=[pl.BlockSpec((NW, bl), lambda i: (0, i))],
      out_specs=pl.BlockSpec((bl, L), lambda i: (i, 0)),
      out_shape=jax.ShapeDtypeStruct((RN, L), jnp.float32),
  )(hists)


def _t3_body(xa_ref, a0_ref, a1_ref, xb_ref, b0_ref, b1_ref,
             oa_ref, ob_ref):
  oa_ref[...] = xa_ref[...] + a0_ref[...] + a1_ref[...]
  ob_ref[...] = xb_ref[...] + b0_ref[...] + b1_ref[...]


def _t3(xa, a0, a1, xb, b0, b1):
  bn = 400
  g = N // bn
  spec = pl.BlockSpec((bn, HD), lambda i: (i, 0))
  tspec = pl.BlockSpec((bn, HD), lambda i: (R * g + i, 0))
  half = jax.ShapeDtypeStruct((N, HD), jnp.float32)
  return pl.pallas_call(
      _t3_body,
      grid=(g,),
      in_specs=[tspec, spec, spec, tspec, spec, spec],
      out_specs=[spec, spec],
      out_shape=[half, half],
  )(xa, a0, a1, xb, b0, b1)


def _t4_body(na0_ref, na1_ref, nb0_ref, nb1_ref, o1a_ref, o1b_ref,
             wra_ref, wrb_ref, wwa_ref, wwb_ref, b_ref, out_ref):
  na = na0_ref[...] + na1_ref[...]
  nb = nb0_ref[...] + nb1_ref[...]
  f32 = jnp.float32
  out_ref[...] = (
      jnp.dot(na, wra_ref[...], preferred_element_type=f32)
      + jnp.dot(nb, wrb_ref[...], preferred_element_type=f32)
      + jnp.dot(o1a_ref[...], wwa_ref[...], preferred_element_type=f32)
      + jnp.dot(o1b_ref[...], wwb_ref[...], preferred_element_type=f32)
      + b_ref[...])


def _t4(na0, na1, nb0, nb1, o1a, o1b, w_rel, w_root, b2d):
  bn = 400
  g = N // bn
  nspec = pl.BlockSpec((bn, HD), lambda i: (i, 0))
  wspec = pl.BlockSpec((HD, D), lambda i: (0, 0))
  return pl.pallas_call(
      _t4_body,
      grid=(g,),
      in_specs=[nspec, nspec, nspec, nspec, nspec, nspec,
                wspec, wspec, wspec, wspec,
                pl.BlockSpec((1, D), lambda i: (0, 0))],
      out_specs=pl.BlockSpec((bn, D), lambda i: (i, 0)),
      out_shape=jax.ShapeDtypeStruct((N, D), jnp.float32),
  )(na0, na1, nb0, nb1, o1a, o1b,
    w_rel[:HD], w_rel[HD:], w_root[:HD], w_root[HD:], b2d)


_MESH = plsc.VectorSubcoreMesh(
    core_axis_name="c", subcore_axis_name="s", num_cores=NC, num_subcores=NS)
_SC_PARAMS = pltpu.CompilerParams(
    needs_layout_passes=False, use_tc_tiling_on_sc=False)


def _a1_body(idx3_hbm, out_hbm, hist_v, ibuf, isem):
  wid = lax.axis_index("s") * NC + lax.axis_index("c")
  nk = _num_chunks_for(wid)
  lane = lax.iota(jnp.int32, L)
  zeros = jnp.zeros((L,), jnp.float32)
  ones = jnp.ones((L,), jnp.float32)

  def zero_body(i, _):
    hist_v[pl.ds(i * L, L)] = zeros
    return 0
  lax.fori_loop(0, RN // L, zero_body, 0)

  def fetch(k, b):
    @pl.when(k < nk)
    def _():
      c = wid + k * NW
      pltpu.async_copy(idx3_hbm.at[c].at[pl.ds(1, 2)], ibuf.at[b],
                       isem.at[b])

  def process(k, b):
    @pl.when(k < nk)
    def _():
      pltpu.make_async_copy(idx3_hbm.at[wid].at[pl.ds(1, 2)], ibuf.at[b],
                            isem.at[b]).wait()

      def blk(i, _):
        dv = ibuf[b, 0, pl.ds(i * L, L)]
        tv = ibuf[b, 1, pl.ds(i * L, L)]
        civ = tv * N + dv
        for j in range(L):
          plsc.addupdate_scatter(hist_v, [civ], ones, mask=(lane == j))
        return 0
      lax.fori_loop(0, K // L, blk, 0)

  fetch(jnp.int32(0), 0)
  fetch(jnp.int32(1), 1)

  def body(kk, _):
    k = kk * 2
    process(k, 0)
    fetch(k + 2, 0)
    process(k + 1, 1)
    fetch(k + 3, 1)
    return 0
  lax.fori_loop(0, (nk + 1) // 2, body, 0)
  pltpu.sync_copy(hist_v, out_hbm.at[wid])


@functools.partial(
    pl.kernel,
    out_type=jax.ShapeDtypeStruct((NW, RN), jnp.float32),
    mesh=_MESH,
    scratch_types=[
        pltpu.VMEM((RN,), jnp.float32),
        pltpu.VMEM((2, 2, K), jnp.int32),
        pltpu.SemaphoreType.DMA((2,)),
    ],
    compiler_params=_SC_PARAMS,
)
def _sc_a1(idx3_hbm, out_hbm, hist_v, ibuf, isem):
  _a1_body(idx3_hbm, out_hbm, hist_v, ibuf, isem)


def _sub_blocks(sid):
  start = sid * 624
  nblk = jnp.where(sid == NS - 1, 40, 39)
  return start, nblk


def _zero_spmem_slice(agg_sh, rows_v, sid):
  def zr(i, _):
    for h in range(HD // L):
      rows_v[i, pl.ds(h * L, L)] = jnp.zeros((L,), jnp.float32)
    return 0
  lax.fori_loop(0, 16, zr, 0)
  start, nblk = _sub_blocks(sid)

  def cp(k, _):
    off = pl.multiple_of(start + k * 16, 8)
    pltpu.sync_copy(rows_v.at[pl.ds(0, 16)], agg_sh.at[pl.ds(off, 16)])
    return 0
  lax.fori_loop(0, nblk, cp, 0)


def _write_out_slice(agg_sh, out_hbm, sid):
  start, nblk = _sub_blocks(sid)

  def cp(k, _):
    off = pl.multiple_of(start + k * 16, 8)
    pltpu.sync_copy(agg_sh.at[pl.ds(off, 16)], out_hbm.at[pl.ds(off, 16)])
    return 0
  lax.fori_loop(0, nblk, cp, 0)


def _agg_pipeline(table_a, table_b, idx3_hbm, wt16_hbm, out_hbm,
                  agg_a, agg_b, ra, rb, ibuf, wbuf, gidx, widx, dstb,
                  isem, wsem, gsa, gsb, ssa, ssb, scaled):
  cid = lax.axis_index("c")
  sid = lax.axis_index("s")
  wid = sid * NC + cid
  nk = _num_chunks_for(wid)

  _zero_spmem_slice(agg_a, ra.at[0], sid)
  _zero_spmem_slice(agg_b, rb.at[0], sid)
  plsc.subcore_barrier()

  def fetch(k, b):
    @pl.when(k < nk)
    def _():
      c = wid + k * NW
      pltpu.async_copy(idx3_hbm.at[c], ibuf.at[b], isem.at[b])

  def gather(k, b):
    @pl.when(k < nk)
    def _():
      pltpu.make_async_copy(
          idx3_hbm.at[wid], ibuf.at[b], isem.at[b]).wait()
      if scaled:
        def gix(i, _):
          sv = ibuf[b, 0, pl.ds(i * L, L)]
          dv = ibuf[b, 1, pl.ds(i * L, L)]
          tv = ibuf[b, 2, pl.ds(i * L, L)]
          tvn = tv * N
          gidx[b, pl.ds(i * L, L)] = tvn + sv
          widx[b, pl.ds(i * L, L)] = tvn + dv
          return 0
        lax.fori_loop(0, K // L, gix, 0)
        src_idx = gidx.at[b]
        pltpu.async_copy(wt16_hbm.at[widx.at[b]], wbuf.at[b], wsem.at[b])
      else:
        src_idx = ibuf.at[b].at[0]

      @pl.when(k >= 2)
      def _():
        pltpu.make_async_copy(
            ra.at[b], agg_a.at[dstb.at[b]], ssa.at[b]).wait()
        pltpu.make_async_copy(
            rb.at[b], agg_b.at[dstb.at[b]], ssb.at[b]).wait()
      pltpu.async_copy(table_a.at[src_idx], ra.at[b], gsa.at[b])
      pltpu.async_copy(table_b.at[src_idx], rb.at[b], gsb.at[b])

  def process(k, b):
    @pl.when(k < nk)
    def _():
      if scaled:
        src_idx = gidx.at[b]
        pltpu.make_async_copy(
            wt16_hbm.at[widx.at[b]], wbuf.at[b], wsem.at[b]).wait()
      else:
        src_idx = ibuf.at[b].at[0]
      pltpu.make_async_copy(table_a.at[src_idx], ra.at[b], gsa.at[b]).wait()
      pltpu.make_async_copy(table_b.at[src_idx], rb.at[b], gsb.at[b]).wait()

      def blk(i, _):
        dstb[b, pl.ds(i * L, L)] = ibuf[b, 1, pl.ds(i * L, L)]
        if scaled:
          for j in range(L):
            e = i * L + j
            w = wbuf[b, e]
            for h in range(HD // L):
              ra[b, e, pl.ds(h * L, L)] = ra[b, e, pl.ds(h * L, L)] * w
              rb[b, e, pl.ds(h * L, L)] = rb[b, e, pl.ds(h * L, L)] * w
        return 0
      lax.fori_loop(0, K // L, blk, 0)

      pltpu.async_copy(ra.at[b], agg_a.at[dstb.at[b]], ssa.at[b], add=True)
      pltpu.async_copy(rb.at[b], agg_b.at[dstb.at[b]], ssb.at[b], add=True)

  fetch(jnp.int32(0), 0)
  fetch(jnp.int32(1), 1)
  gather(jnp.int32(0), 0)

  def body(kk, _):
    k = kk * 2
    process(k, 0)
    fetch(k + 2, 0)
    gather(k + 1, 1)
    process(k + 1, 1)
    fetch(k + 3, 1)
    gather(k + 2, 0)
    return 0
  lax.fori_loop(0, (nk + 1) // 2, body, 0)

  @pl.when(nk >= 2)
  def _():
    for b in range(2):
      pltpu.make_async_copy(
          ra.at[b], agg_a.at[dstb.at[b]], ssa.at[b]).wait()
      pltpu.make_async_copy(
          rb.at[b], agg_b.at[dstb.at[b]], ssb.at[b]).wait()

  @pl.when(nk == 1)
  def _():
    pltpu.make_async_copy(
        ra.at[0], agg_a.at[dstb.at[0]], ssa.at[0]).wait()
    pltpu.make_async_copy(
        rb.at[0], agg_b.at[dstb.at[0]], ssb.at[0]).wait()

  plsc.subcore_barrier()
  _write_out_slice(agg_a, out_hbm.at[0].at[cid], sid)
  _write_out_slice(agg_b, out_hbm.at[1].at[cid], sid)


_AGG_SCRATCH = [
    pltpu.VMEM_SHARED((N, HD), jnp.float32),
    pltpu.VMEM_SHARED((N, HD), jnp.float32),
    pltpu.VMEM((2, K, HD), jnp.float32),
    pltpu.VMEM((2, K, HD), jnp.float32),
    pltpu.VMEM((2, 3, K), jnp.int32),
    pltpu.VMEM((2, K, L), jnp.float32),
    pltpu.VMEM((2, K), jnp.int32),
    pltpu.VMEM((2, K), jnp.int32),
    pltpu.VMEM((2, K), jnp.int32),
    pltpu.SemaphoreType.DMA((2,)),
    pltpu.SemaphoreType.DMA((2,)),
    pltpu.SemaphoreType.DMA((2,)),
    pltpu.SemaphoreType.DMA((2,)),
    pltpu.SemaphoreType.DMA((2,)),
    pltpu.SemaphoreType.DMA((2,)),
]


@functools.partial(
    pl.kernel,
    out_type=jax.ShapeDtypeStruct((2, NC, N, HD), jnp.float32),
    mesh=_MESH,
    scratch_types=_AGG_SCRATCH,
    compiler_params=_SC_PARAMS,
)
def _sc_a2(ta, tb, idx3, wt16, out, agg_a, agg_b, ra, rb, ibuf, wbuf,
           gidx, widx, dstb, isem, wsem, gsa, gsb, ssa, ssb):
  _agg_pipeline(ta, tb, idx3, wt16, out, agg_a, agg_b, ra, rb, ibuf, wbuf,
                gidx, widx, dstb, isem, wsem, gsa, gsb, ssa, ssb,
                scaled=True)


@functools.partial(
    pl.kernel,
    out_type=jax.ShapeDtypeStruct((2, NC, N, HD), jnp.float32),
    mesh=_MESH,
    scratch_types=_AGG_SCRATCH,
    compiler_params=_SC_PARAMS,
)
def _sc_b(ta, tb, idx3, wt16, out, agg_a, agg_b, ra, rb, ibuf, wbuf,
          gidx, widx, dstb, isem, wsem, gsa, gsb, ssa, ssb):
  _agg_pipeline(ta, tb, idx3, wt16, out, agg_a, agg_b, ra, rb, ibuf, wbuf,
                gidx, widx, dstb, isem, wsem, gsa, gsb, ssa, ssb,
                scaled=False)


def kernel(node_features, edge_index, edge_norm, edge_type, basis, comp,
           root_w, root_b, gc_w_rel, gc_w_root, gc_b):
  del edge_norm
  src = edge_index[0].astype(jnp.int32)
  dst = edge_index[1].astype(jnp.int32)
  et = edge_type.astype(jnp.int32)
  idx3 = jnp.stack([src, dst, et]).reshape(3, CHUNKS, K).transpose(1, 0, 2)

  w8 = _t0(comp, basis.reshape(NBASES, D * D))
  w_all = jnp.concatenate(
      [w8.reshape(R, D, D), root_w[None]], axis=0)

  tbl_a, tbl_b = _t1(node_features, w_all, root_b.reshape(1, D))

  hists = _sc_a1(idx3)
  wt16 = _t2(hists)

  agg = _sc_a2(tbl_a, tbl_b, idx3, wt16)
  o1a, o1b = _t3(tbl_a, agg[0, 0], agg[0, 1],
                 tbl_b, agg[1, 0], agg[1, 1])

  neigh = _sc_b(o1a, o1b, idx3, wt16)
  out2 = _t4(neigh[0, 0], neigh[0, 1], neigh[1, 0], neigh[1, 1],
             o1a, o1b, gc_w_rel, gc_w_root, gc_b.reshape(1, D))
  return out2


# speedup vs baseline: 1.5054x; 1.4524x over previous
import functools

import jax
import jax.numpy as jnp
from jax import lax
from jax.experimental import pallas as pl
from jax.experimental.pallas import tpu as pltpu
from jax.experimental.pallas import tpu_sc as plsc

N = 10000
E = 320000
D = 128
HD = D // 2
R = 8
NBASES = 30
RN = R * N

NC = 2
NS = 16
NW = NC * NS
L = 16

K = 128
CHUNKS = E // K

_GDN = lax.GatherDimensionNumbers(
    offset_dims=(), collapsed_slice_dims=(0,), start_index_map=(0,))


def _bcast_lane(vec, j):
  idx = jnp.full((L, 1), j, jnp.int32)
  return lax.gather(vec, idx, _GDN, slice_sizes=(1,),
                    mode=lax.GatherScatterMode.PROMISE_IN_BOUNDS)


def _num_chunks_for(wid):
  return (CHUNKS - wid + NW - 1) // NW


def _t0_body(comp_ref, basis_ref, out_ref):
  out_ref[...] = jnp.dot(comp_ref[...], basis_ref[...],
                         preferred_element_type=jnp.float32)


def _t0(comp, basis2d):
  g = basis2d.shape[1] // 2048
  return pl.pallas_call(
      _t0_body,
      grid=(g,),
      in_specs=[
          pl.BlockSpec((R, NBASES), lambda i: (0, 0)),
          pl.BlockSpec((NBASES, 2048), lambda i: (0, i)),
      ],
      out_specs=pl.BlockSpec((R, 2048), lambda i: (0, i)),
      out_shape=jax.ShapeDtypeStruct((R, basis2d.shape[1]), jnp.float32),
  )(comp, basis2d)


def _t1_body(x_ref, w_ref, b_ref, out_ref):
  o = jnp.dot(x_ref[...], w_ref[0], preferred_element_type=jnp.float32)
  bias = jnp.where(pl.program_id(1) == R, b_ref[...], 0.0)
  out_ref[...] = o + bias


def _t1(x, w_all, root_b2d):
  bn = 1000
  g = N // bn
  return pl.pallas_call(
      _t1_body,
      grid=(g, R + 1),
      in_specs=[
          pl.BlockSpec((bn, D), lambda i, r: (i, 0)),
          pl.BlockSpec((1, D, D), lambda i, r: (r, 0, 0)),
          pl.BlockSpec((1, D), lambda i, r: (0, 0)),
      ],
      out_specs=pl.BlockSpec((bn, D), lambda i, r: (r * g + i, 0)),
      out_shape=jax.ShapeDtypeStruct(((R + 1) * N, D), jnp.float32),
  )(x, w_all, root_b2d)


def _t2_body(h_ref, out_ref):
  s = jnp.sum(h_ref[...], axis=0, keepdims=True)
  w = 1.0 / jnp.maximum(s, 1.0)
  out_ref[...] = jnp.broadcast_to(w.reshape(-1, 1), out_ref.shape)


def _t2(hists):
  bl = 3200
  g = RN // bl
  return pl.pallas_call(
      _t2_body,
      grid=(g,),
      in_specs=[pl.BlockSpec((NW, bl), lambda i: (0, i))],
      out_specs=pl.BlockSpec((bl, L), lambda i: (i, 0)),
      out_shape=jax.ShapeDtypeStruct((RN, L), jnp.float32),
  )(hists)


def _t3_body(xr_ref, a0_ref, a1_ref, out_ref):
  out_ref[...] = xr_ref[...] + a0_ref[...] + a1_ref[...]


def _t3(table, a0, a1):
  bn = 1000
  g = N // bn
  spec = pl.BlockSpec((bn, D), lambda i: (i, 0))
  tspec = pl.BlockSpec((bn, D), lambda i: (R * g + i, 0))
  return pl.pallas_call(
      _t3_body,
      grid=(g,),
      in_specs=[tspec, spec, spec],
      out_specs=spec,
      out_shape=jax.ShapeDtypeStruct((N, D), jnp.float32),
  )(table, a0, a1)


def _t4_body(n0_ref, n1_ref, o1_ref, wr_ref, ww_ref, b_ref, out_ref):
  neigh = n0_ref[...] + n1_ref[...]
  f32 = jnp.float32
  out_ref[...] = (
      jnp.dot(neigh, wr_ref[...], preferred_element_type=f32)
      + jnp.dot(o1_ref[...], ww_ref[...], preferred_element_type=f32)
      + b_ref[...])


def _t4(n0, n1, o1, w_rel, w_root, b2d):
  bn = 1000
  g = N // bn
  spec = pl.BlockSpec((bn, D), lambda i: (i, 0))
  wspec = pl.BlockSpec((D, D), lambda i: (0, 0))
  return pl.pallas_call(
      _t4_body,
      grid=(g,),
      in_specs=[spec, spec, spec, wspec, wspec,
                pl.BlockSpec((1, D), lambda i: (0, 0))],
      out_specs=spec,
      out_shape=jax.ShapeDtypeStruct((N, D), jnp.float32),
  )(n0, n1, o1, w_rel, w_root, b2d)


_MESH = plsc.VectorSubcoreMesh(
    core_axis_name="c", subcore_axis_name="s", num_cores=NC, num_subcores=NS)
_SC_PARAMS = pltpu.CompilerParams(
    needs_layout_passes=False, use_tc_tiling_on_sc=False)


def _a1_body(dst_hbm, et_hbm, out_hbm, hist_v, dbuf, tbuf, dsem, tsem):
  wid = lax.axis_index("s") * NC + lax.axis_index("c")
  nk = _num_chunks_for(wid)
  lane = lax.iota(jnp.int32, L)
  zeros = jnp.zeros((L,), jnp.float32)
  ones = jnp.ones((L,), jnp.float32)

  def zero_body(i, _):
    hist_v[pl.ds(i * L, L)] = zeros
    return 0
  lax.fori_loop(0, RN // L, zero_body, 0)

  def fetch(k, b):
    @pl.when(k < nk)
    def _():
      c = wid + k * NW
      pltpu.async_copy(dst_hbm.at[pl.ds(c * K, K)], dbuf.at[b], dsem.at[b])
      pltpu.async_copy(et_hbm.at[pl.ds(c * K, K)], tbuf.at[b], tsem.at[b])

  def process(k, b):
    @pl.when(k < nk)
    def _():
      pltpu.make_async_copy(dst_hbm.at[pl.ds(0, K)], dbuf.at[b],
                            dsem.at[b]).wait()
      pltpu.make_async_copy(et_hbm.at[pl.ds(0, K)], tbuf.at[b],
                            tsem.at[b]).wait()

      def blk(i, _):
        dv = dbuf[b, pl.ds(i * L, L)]
        tv = tbuf[b, pl.ds(i * L, L)]
        civ = tv * N + dv
        for j in range(L):
          plsc.addupdate_scatter(hist_v, [civ], ones, mask=(lane == j))
        return 0
      lax.fori_loop(0, K // L, blk, 0)

  fetch(jnp.int32(0), 0)
  fetch(jnp.int32(1), 1)

  def body(kk, _):
    k = kk * 2
    process(k, 0)
    fetch(k + 2, 0)
    process(k + 1, 1)
    fetch(k + 3, 1)
    return 0
  lax.fori_loop(0, (nk + 1) // 2, body, 0)
  pltpu.sync_copy(hist_v, out_hbm.at[wid])


@functools.partial(
    pl.kernel,
    out_type=jax.ShapeDtypeStruct((NW, RN), jnp.float32),
    mesh=_MESH,
    scratch_types=[
        pltpu.VMEM((RN,), jnp.float32),
        pltpu.VMEM((2, K), jnp.int32),
        pltpu.VMEM((2, K), jnp.int32),
        pltpu.SemaphoreType.DMA((2,)),
        pltpu.SemaphoreType.DMA((2,)),
    ],
    compiler_params=_SC_PARAMS,
)
def _sc_a1(dst_hbm, et_hbm, out_hbm, hist_v, dbuf, tbuf, dsem, tsem):
  _a1_body(dst_hbm, et_hbm, out_hbm, hist_v, dbuf, tbuf, dsem, tsem)


def _sub_blocks(sid):
  start = sid * 624
  nblk = jnp.where(sid == NS - 1, 40, 39)
  return start, nblk


def _zero_spmem_slice(agg_sh, rows_v, sid):
  def zr(i, _):
    for h in range(HD // L):
      rows_v[i, pl.ds(h * L, L)] = jnp.zeros((L,), jnp.float32)
    return 0
  lax.fori_loop(0, 16, zr, 0)
  start, nblk = _sub_blocks(sid)

  def cp(k, _):
    off = pl.multiple_of(start + k * 16, 8)
    pltpu.sync_copy(rows_v.at[pl.ds(0, 16)], agg_sh.at[pl.ds(off, 16)])
    return 0
  lax.fori_loop(0, nblk, cp, 0)


def _write_out_slice(agg_sh, out_hbm, sid):
  start, nblk = _sub_blocks(sid)

  def cp(k, _):
    off = pl.multiple_of(start + k * 16, 8)
    pltpu.sync_copy(agg_sh.at[pl.ds(off, 16)], out_hbm.at[pl.ds(off, 16)])
    return 0
  lax.fori_loop(0, nblk, cp, 0)


def _agg_pipeline(table, src_hbm, dst_hbm, et_hbm, wt16_hbm, out_hbm,
                  agg, rows, sbuf, dbuf, tbuf, wbuf, gidx, widx, dstb,
                  ssem_i, dsem_i, tsem_i, wsem, gsem, ssem, scaled):
  cid = lax.axis_index("c")
  sid = lax.axis_index("s")
  wid = sid * NC + cid
  nk = _num_chunks_for(wid)

  _zero_spmem_slice(agg, rows.at[0], sid)
  plsc.subcore_barrier()

  def fetch(k, b):
    @pl.when(k < nk)
    def _():
      c = wid + k * NW
      pltpu.async_copy(src_hbm.at[pl.ds(c * K, K)], sbuf.at[b],
                       ssem_i.at[b])
      pltpu.async_copy(dst_hbm.at[pl.ds(c * K, K)], dbuf.at[b],
                       dsem_i.at[b])
      if scaled:
        pltpu.async_copy(et_hbm.at[pl.ds(c * K, K)], tbuf.at[b],
                         tsem_i.at[b])

  def gather(k, b):
    @pl.when(k < nk)
    def _():
      pltpu.make_async_copy(src_hbm.at[pl.ds(0, K)], sbuf.at[b],
                            ssem_i.at[b]).wait()
      if scaled:
        pltpu.make_async_copy(dst_hbm.at[pl.ds(0, K)], dbuf.at[b],
                              dsem_i.at[b]).wait()
        pltpu.make_async_copy(et_hbm.at[pl.ds(0, K)], tbuf.at[b],
                              tsem_i.at[b]).wait()

        def gix(i, _):
          sv = sbuf[b, pl.ds(i * L, L)]
          dv = dbuf[b, pl.ds(i * L, L)]
          tv = tbuf[b, pl.ds(i * L, L)]
          tvn = tv * N
          gidx[b, pl.ds(i * L, L)] = tvn + sv
          widx[b, pl.ds(i * L, L)] = tvn + dv
          return 0
        lax.fori_loop(0, K // L, gix, 0)
        src_idx = gidx.at[b]
        pltpu.async_copy(wt16_hbm.at[widx.at[b]], wbuf.at[b], wsem.at[b])
      else:
        src_idx = sbuf.at[b]

      @pl.when(k >= 2)
      def _():
        pltpu.make_async_copy(
            rows.at[b], agg.at[dstb.at[b]], ssem.at[b]).wait()
      pltpu.async_copy(table.at[src_idx], rows.at[b], gsem.at[b])

  def process(k, b):
    @pl.when(k < nk)
    def _():
      if scaled:
        src_idx = gidx.at[b]
        pltpu.make_async_copy(
            wt16_hbm.at[widx.at[b]], wbuf.at[b], wsem.at[b]).wait()
      else:
        src_idx = sbuf.at[b]
        pltpu.make_async_copy(dst_hbm.at[pl.ds(0, K)], dbuf.at[b],
                              dsem_i.at[b]).wait()
      pltpu.make_async_copy(table.at[src_idx], rows.at[b], gsem.at[b]).wait()

      def blk(i, _):
        dstb[b, pl.ds(i * L, L)] = dbuf[b, pl.ds(i * L, L)]
        if scaled:
          for j in range(L):
            e = i * L + j
            w = wbuf[b, e]
            for h in range(D // L):
              rows[b, e, pl.ds(h * L, L)] = rows[b, e, pl.ds(h * L, L)] * w
        return 0
      lax.fori_loop(0, K // L, blk, 0)

      pltpu.async_copy(rows.at[b], agg.at[dstb.at[b]], ssem.at[b],
                       add=True)

  fetch(jnp.int32(0), 0)
  fetch(jnp.int32(1), 1)
  gather(jnp.int32(0), 0)

  def body(kk, _):
    k = kk * 2
    process(k, 0)
    fetch(k + 2, 0)
    gather(k + 1, 1)
    process(k + 1, 1)
    fetch(k + 3, 1)
    gather(k + 2, 0)
    return 0
  lax.fori_loop(0, (nk + 1) // 2, body, 0)

  @pl.when(nk >= 2)
  def _():
    for b in range(2):
      pltpu.make_async_copy(
          rows.at[b], agg.at[dstb.at[b]], ssem.at[b]).wait()

  @pl.when(nk == 1)
  def _():
    pltpu.make_async_copy(
        rows.at[0], agg.at[dstb.at[0]], ssem.at[0]).wait()

  plsc.subcore_barrier()
  _write_out_slice(agg, out_hbm.at[cid], sid)


_AGG_SCRATCH = [
    pltpu.VMEM_SHARED((N, D), jnp.float32),
    pltpu.VMEM((2, K, D), jnp.float32),
    pltpu.VMEM((2, K), jnp.int32),
    pltpu.VMEM((2, K), jnp.int32),
    pltpu.VMEM((2, K), jnp.int32),
    pltpu.VMEM((2, K, L), jnp.float32),
    pltpu.VMEM((2, K), jnp.int32),
    pltpu.VMEM((2, K), jnp.int32),
    pltpu.VMEM((2, K), jnp.int32),
    pltpu.SemaphoreType.DMA((2,)),
    pltpu.SemaphoreType.DMA((2,)),
    pltpu.SemaphoreType.DMA((2,)),
    pltpu.SemaphoreType.DMA((2,)),
    pltpu.SemaphoreType.DMA((2,)),
    pltpu.SemaphoreType.DMA((2,)),
]


@functools.partial(
    pl.kernel,
    out_type=jax.ShapeDtypeStruct((NC, N, D), jnp.float32),
    mesh=_MESH,
    scratch_types=_AGG_SCRATCH,
    compiler_params=_SC_PARAMS,
)
def _sc_a2(table, src, dst, et, wt16, out, agg, rows,
           sbuf, dbuf, tbuf, wbuf, gidx, widx, dstb,
           ssem_i, dsem_i, tsem_i, wsem, gsem, ssem):
  _agg_pipeline(table, src, dst, et, wt16, out, agg, rows,
                sbuf, dbuf, tbuf, wbuf, gidx, widx, dstb,
                ssem_i, dsem_i, tsem_i, wsem, gsem, ssem, scaled=True)


@functools.partial(
    pl.kernel,
    out_type=jax.ShapeDtypeStruct((NC, N, D), jnp.float32),
    mesh=_MESH,
    scratch_types=_AGG_SCRATCH,
    compiler_params=_SC_PARAMS,
)
def _sc_b(table, src, dst, et, wt16, out, agg, rows,
          sbuf, dbuf, tbuf, wbuf, gidx, widx, dstb,
          ssem_i, dsem_i, tsem_i, wsem, gsem, ssem):
  _agg_pipeline(table, src, dst, et, wt16, out, agg, rows,
                sbuf, dbuf, tbuf, wbuf, gidx, widx, dstb,
                ssem_i, dsem_i, tsem_i, wsem, gsem, ssem, scaled=False)


def kernel(node_features, edge_index, edge_norm, edge_type, basis, comp,
           root_w, root_b, gc_w_rel, gc_w_root, gc_b):
  del edge_norm
  src = edge_index[0].astype(jnp.int32)
  dst = edge_index[1].astype(jnp.int32)
  et = edge_type.astype(jnp.int32)

  w8 = _t0(comp, basis.reshape(NBASES, D * D))
  w_all = jnp.concatenate(
      [w8.reshape(R, D, D), root_w[None]], axis=0)

  table = _t1(node_features, w_all, root_b.reshape(1, D))

  hists = _sc_a1(dst, et)
  wt16 = _t2(hists)

  agg = _sc_a2(table, src, dst, et, wt16)
  out1 = _t3(table, agg[0], agg[1])

  neigh = _sc_b(out1, src, dst, et, wt16)
  out2 = _t4(neigh[0], neigh[1], out1, gc_w_rel, gc_w_root,
             gc_b.reshape(1, D))
  return out2


# speedup vs baseline: 1.5054x; 1.0000x over previous
import functools

import jax
import jax.numpy as jnp
from jax import lax
from jax.experimental import pallas as pl
from jax.experimental.pallas import tpu as pltpu
from jax.experimental.pallas import tpu_sc as plsc

N = 10000
E = 320000
D = 128
HD = D // 2
R = 8
NBASES = 30
RN = R * N

NC = 2
NS = 16
NW = NC * NS
L = 16

K = 128
CHUNKS = E // K

_GDN = lax.GatherDimensionNumbers(
    offset_dims=(), collapsed_slice_dims=(0,), start_index_map=(0,))


def _bcast_lane(vec, j):
  idx = jnp.full((L, 1), j, jnp.int32)
  return lax.gather(vec, idx, _GDN, slice_sizes=(1,),
                    mode=lax.GatherScatterMode.PROMISE_IN_BOUNDS)


def _num_chunks_for(wid):
  return (CHUNKS - wid + NW - 1) // NW


def _t0_body(comp_ref, basis_ref, out_ref):
  out_ref[...] = jnp.dot(comp_ref[...], basis_ref[...],
                         preferred_element_type=jnp.float32)


def _t0(comp, basis2d):
  g = basis2d.shape[1] // 2048
  return pl.pallas_call(
      _t0_body,
      grid=(g,),
      in_specs=[
          pl.BlockSpec((R, NBASES), lambda i: (0, 0)),
          pl.BlockSpec((NBASES, 2048), lambda i: (0, i)),
      ],
      out_specs=pl.BlockSpec((R, 2048), lambda i: (0, i)),
      out_shape=jax.ShapeDtypeStruct((R, basis2d.shape[1]), jnp.float32),
  )(comp, basis2d)


def _t1_body(x_ref, w_ref, b_ref, out_ref):
  o = jnp.dot(x_ref[...], w_ref[0], preferred_element_type=jnp.float32)
  bias = jnp.where(pl.program_id(1) == R, b_ref[...], 0.0)
  out_ref[...] = o + bias


def _t1(x, w_all, root_b2d):
  bn = 1000
  g = N // bn
  return pl.pallas_call(
      _t1_body,
      grid=(g, R + 1),
      in_specs=[
          pl.BlockSpec((bn, D), lambda i, r: (i, 0)),
          pl.BlockSpec((1, D, D), lambda i, r: (r, 0, 0)),
          pl.BlockSpec((1, D), lambda i, r: (0, 0)),
      ],
      out_specs=pl.BlockSpec((bn, D), lambda i, r: (r * g + i, 0)),
      out_shape=jax.ShapeDtypeStruct(((R + 1) * N, D), jnp.float32),
  )(x, w_all, root_b2d)


def _t2_body(h_ref, out_ref):
  s = jnp.sum(h_ref[...], axis=0, keepdims=True)
  w = 1.0 / jnp.maximum(s, 1.0)
  out_ref[...] = jnp.broadcast_to(w.reshape(-1, 1), out_ref.shape)


def _t2(hists):
  bl = 3200
  g = RN // bl
  return pl.pallas_call(
      _t2_body,
      grid=(g,),
      in_specs=[pl.BlockSpec((NW, bl), lambda i: (0, i))],
      out_specs=pl.BlockSpec((bl, L), lambda i: (i, 0)),
      out_shape=jax.ShapeDtypeStruct((RN, L), jnp.float32),
  )(hists)


def _t3_body(xr_ref, a0_ref, a1_ref, out_ref):
  out_ref[...] = xr_ref[...] + a0_ref[...] + a1_ref[...]


def _t3(table, a0, a1):
  bn = 1000
  g = N // bn
  spec = pl.BlockSpec((bn, D), lambda i: (i, 0))
  tspec = pl.BlockSpec((bn, D), lambda i: (R * g + i, 0))
  return pl.pallas_call(
      _t3_body,
      grid=(g,),
      in_specs=[tspec, spec, spec],
      out_specs=spec,
      out_shape=jax.ShapeDtypeStruct((N, D), jnp.float32),
  )(table, a0, a1)


def _t4_body(n0_ref, n1_ref, o1_ref, wr_ref, ww_ref, b_ref, out_ref):
  neigh = n0_ref[...] + n1_ref[...]
  f32 = jnp.float32
  out_ref[...] = (
      jnp.dot(neigh, wr_ref[...], preferred_element_type=f32)
      + jnp.dot(o1_ref[...], ww_ref[...], preferred_element_type=f32)
      + b_ref[...])


def _t4(n0, n1, o1, w_rel, w_root, b2d):
  bn = 1000
  g = N // bn
  spec = pl.BlockSpec((bn, D), lambda i: (i, 0))
  wspec = pl.BlockSpec((D, D), lambda i: (0, 0))
  return pl.pallas_call(
      _t4_body,
      grid=(g,),
      in_specs=[spec, spec, spec, wspec, wspec,
                pl.BlockSpec((1, D), lambda i: (0, 0))],
      out_specs=spec,
      out_shape=jax.ShapeDtypeStruct((N, D), jnp.float32),
  )(n0, n1, o1, w_rel, w_root, b2d)


_MESH = plsc.VectorSubcoreMesh(
    core_axis_name="c", subcore_axis_name="s", num_cores=NC, num_subcores=NS)
_SC_PARAMS = pltpu.CompilerParams(
    needs_layout_passes=False, use_tc_tiling_on_sc=False)


def _a1_body(dst_hbm, et_hbm, out_hbm, hist_v, dbuf, tbuf, dsem, tsem):
  wid = lax.axis_index("s") * NC + lax.axis_index("c")
  nk = _num_chunks_for(wid)
  lane = lax.iota(jnp.int32, L)
  zeros = jnp.zeros((L,), jnp.float32)
  ones = jnp.ones((L,), jnp.float32)

  def zero_body(i, _):
    hist_v[pl.ds(i * L, L)] = zeros
    return 0
  lax.fori_loop(0, RN // L, zero_body, 0)

  def fetch(k, b):
    @pl.when(k < nk)
    def _():
      c = wid + k * NW
      pltpu.async_copy(dst_hbm.at[pl.ds(c * K, K)], dbuf.at[b], dsem.at[b])
      pltpu.async_copy(et_hbm.at[pl.ds(c * K, K)], tbuf.at[b], tsem.at[b])

  def process(k, b):
    @pl.when(k < nk)
    def _():
      pltpu.make_async_copy(dst_hbm.at[pl.ds(0, K)], dbuf.at[b],
                            dsem.at[b]).wait()
      pltpu.make_async_copy(et_hbm.at[pl.ds(0, K)], tbuf.at[b],
                            tsem.at[b]).wait()

      def blk(i, _):
        dv = dbuf[b, pl.ds(i * L, L)]
        tv = tbuf[b, pl.ds(i * L, L)]
        civ = tv * N + dv
        for j in range(L):
          plsc.addupdate_scatter(hist_v, [civ], ones, mask=(lane == j))
        return 0
      lax.fori_loop(0, K // L, blk, 0)

  fetch(jnp.int32(0), 0)
  fetch(jnp.int32(1), 1)

  def body(kk, _):
    k = kk * 2
    process(k, 0)
    fetch(k + 2, 0)
    process(k + 1, 1)
    fetch(k + 3, 1)
    return 0
  lax.fori_loop(0, (nk + 1) // 2, body, 0)
  pltpu.sync_copy(hist_v, out_hbm.at[wid])


@functools.partial(
    pl.kernel,
    out_type=jax.ShapeDtypeStruct((NW, RN), jnp.float32),
    mesh=_MESH,
    scratch_types=[
        pltpu.VMEM((RN,), jnp.float32),
        pltpu.VMEM((2, K), jnp.int32),
        pltpu.VMEM((2, K), jnp.int32),
        pltpu.SemaphoreType.DMA((2,)),
        pltpu.SemaphoreType.DMA((2,)),
    ],
    compiler_params=_SC_PARAMS,
)
def _sc_a1(dst_hbm, et_hbm, out_hbm, hist_v, dbuf, tbuf, dsem, tsem):
  _a1_body(dst_hbm, et_hbm, out_hbm, hist_v, dbuf, tbuf, dsem, tsem)


def _sub_blocks(sid):
  start = sid * 624
  nblk = jnp.where(sid == NS - 1, 40, 39)
  return start, nblk


def _zero_spmem_slice(agg_sh, rows_v, sid):
  def zr(i, _):
    for h in range(rows_v.shape[-1] // L):
      rows_v[i, pl.ds(h * L, L)] = jnp.zeros((L,), jnp.float32)
    return 0
  lax.fori_loop(0, 16, zr, 0)
  start, nblk = _sub_blocks(sid)

  def cp(k, _):
    off = pl.multiple_of(start + k * 16, 8)
    pltpu.sync_copy(rows_v.at[pl.ds(0, 16)], agg_sh.at[pl.ds(off, 16)])
    return 0
  lax.fori_loop(0, nblk, cp, 0)


def _write_out_slice(agg_sh, out_hbm, sid):
  start, nblk = _sub_blocks(sid)

  def cp(k, _):
    off = pl.multiple_of(start + k * 16, 8)
    pltpu.sync_copy(agg_sh.at[pl.ds(off, 16)], out_hbm.at[pl.ds(off, 16)])
    return 0
  lax.fori_loop(0, nblk, cp, 0)


def _agg_pipeline(table, src_hbm, dst_hbm, et_hbm, wt16_hbm, out_hbm,
                  agg, rows, sbuf, dbuf, tbuf, wbuf, gidx, widx, dstb,
                  ssem_i, dsem_i, tsem_i, wsem, gsem, ssem, scaled):
  cid = lax.axis_index("c")
  sid = lax.axis_index("s")
  wid = sid * NC + cid
  nk = _num_chunks_for(wid)

  _zero_spmem_slice(agg, rows.at[0], sid)
  plsc.subcore_barrier()

  def fetch(k, b):
    @pl.when(k < nk)
    def _():
      c = wid + k * NW
      pltpu.async_copy(src_hbm.at[pl.ds(c * K, K)], sbuf.at[b],
                       ssem_i.at[b])
      pltpu.async_copy(dst_hbm.at[pl.ds(c * K, K)], dbuf.at[b],
                       dsem_i.at[b])
      if scaled:
        pltpu.async_copy(et_hbm.at[pl.ds(c * K, K)], tbuf.at[b],
                         tsem_i.at[b])

  def gather(k, b):
    @pl.when(k < nk)
    def _():
      pltpu.make_async_copy(src_hbm.at[pl.ds(0, K)], sbuf.at[b],
                            ssem_i.at[b]).wait()
      if scaled:
        pltpu.make_async_copy(dst_hbm.at[pl.ds(0, K)], dbuf.at[b],
                              dsem_i.at[b]).wait()
        pltpu.make_async_copy(et_hbm.at[pl.ds(0, K)], tbuf.at[b],
                              tsem_i.at[b]).wait()

        def gix(i, _):
          sv = sbuf[b, pl.ds(i * L, L)]
          dv = dbuf[b, pl.ds(i * L, L)]
          tv = tbuf[b, pl.ds(i * L, L)]
          tvn = tv * N
          gidx[b, pl.ds(i * L, L)] = tvn + sv
          widx[b, pl.ds(i * L, L)] = tvn + dv
          return 0
        lax.fori_loop(0, K // L, gix, 0)
        src_idx = gidx.at[b]
        pltpu.async_copy(wt16_hbm.at[widx.at[b]], wbuf.at[b], wsem.at[b])
      else:
        src_idx = sbuf.at[b]

      @pl.when(k >= 2)
      def _():
        pltpu.make_async_copy(
            rows.at[b], agg.at[dstb.at[b]], ssem.at[b]).wait()
      pltpu.async_copy(table.at[src_idx], rows.at[b], gsem.at[b])

  def process(k, b):
    @pl.when(k < nk)
    def _():
      if scaled:
        src_idx = gidx.at[b]
        pltpu.make_async_copy(
            wt16_hbm.at[widx.at[b]], wbuf.at[b], wsem.at[b]).wait()
      else:
        src_idx = sbuf.at[b]
        pltpu.make_async_copy(dst_hbm.at[pl.ds(0, K)], dbuf.at[b],
                              dsem_i.at[b]).wait()
      pltpu.make_async_copy(table.at[src_idx], rows.at[b], gsem.at[b]).wait()

      def blk(i, _):
        dstb[b, pl.ds(i * L, L)] = dbuf[b, pl.ds(i * L, L)]
        if scaled:
          for j in range(L):
            e = i * L + j
            w = wbuf[b, e]
            for h in range(D // L):
              rows[b, e, pl.ds(h * L, L)] = rows[b, e, pl.ds(h * L, L)] * w
        return 0
      lax.fori_loop(0, K // L, blk, 0)

      pltpu.async_copy(rows.at[b], agg.at[dstb.at[b]], ssem.at[b],
                       add=True)

  fetch(jnp.int32(0), 0)
  fetch(jnp.int32(1), 1)
  gather(jnp.int32(0), 0)

  def body(kk, _):
    k = kk * 2
    process(k, 0)
    fetch(k + 2, 0)
    gather(k + 1, 1)
    process(k + 1, 1)
    fetch(k + 3, 1)
    gather(k + 2, 0)
    return 0
  lax.fori_loop(0, (nk + 1) // 2, body, 0)

  @pl.when(nk >= 2)
  def _():
    for b in range(2):
      pltpu.make_async_copy(
          rows.at[b], agg.at[dstb.at[b]], ssem.at[b]).wait()

  @pl.when(nk == 1)
  def _():
    pltpu.make_async_copy(
        rows.at[0], agg.at[dstb.at[0]], ssem.at[0]).wait()

  plsc.subcore_barrier()
  _write_out_slice(agg, out_hbm.at[cid], sid)


_AGG_SCRATCH = [
    pltpu.VMEM_SHARED((N, D), jnp.float32),
    pltpu.VMEM((2, K, D), jnp.float32),
    pltpu.VMEM((2, K), jnp.int32),
    pltpu.VMEM((2, K), jnp.int32),
    pltpu.VMEM((2, K), jnp.int32),
    pltpu.VMEM((2, K, L), jnp.float32),
    pltpu.VMEM((2, K), jnp.int32),
    pltpu.VMEM((2, K), jnp.int32),
    pltpu.VMEM((2, K), jnp.int32),
    pltpu.SemaphoreType.DMA((2,)),
    pltpu.SemaphoreType.DMA((2,)),
    pltpu.SemaphoreType.DMA((2,)),
    pltpu.SemaphoreType.DMA((2,)),
    pltpu.SemaphoreType.DMA((2,)),
    pltpu.SemaphoreType.DMA((2,)),
]


@functools.partial(
    pl.kernel,
    out_type=jax.ShapeDtypeStruct((NC, N, D), jnp.float32),
    mesh=_MESH,
    scratch_types=_AGG_SCRATCH,
    compiler_params=_SC_PARAMS,
)
def _sc_a2(table, src, dst, et, wt16, out, agg, rows,
           sbuf, dbuf, tbuf, wbuf, gidx, widx, dstb,
           ssem_i, dsem_i, tsem_i, wsem, gsem, ssem):
  _agg_pipeline(table, src, dst, et, wt16, out, agg, rows,
                sbuf, dbuf, tbuf, wbuf, gidx, widx, dstb,
                ssem_i, dsem_i, tsem_i, wsem, gsem, ssem, scaled=True)


@functools.partial(
    pl.kernel,
    out_type=jax.ShapeDtypeStruct((NC, N, D), jnp.float32),
    mesh=_MESH,
    scratch_types=_AGG_SCRATCH,
    compiler_params=_SC_PARAMS,
)
def _sc_b(table, src, dst, et, wt16, out, agg, rows,
          sbuf, dbuf, tbuf, wbuf, gidx, widx, dstb,
          ssem_i, dsem_i, tsem_i, wsem, gsem, ssem):
  _agg_pipeline(table, src, dst, et, wt16, out, agg, rows,
                sbuf, dbuf, tbuf, wbuf, gidx, widx, dstb,
                ssem_i, dsem_i, tsem_i, wsem, gsem, ssem, scaled=False)


def kernel(node_features, edge_index, edge_norm, edge_type, basis, comp,
           root_w, root_b, gc_w_rel, gc_w_root, gc_b):
  del edge_norm
  src = edge_index[0].astype(jnp.int32)
  dst = edge_index[1].astype(jnp.int32)
  et = edge_type.astype(jnp.int32)

  w8 = _t0(comp, basis.reshape(NBASES, D * D))
  w_all = jnp.concatenate(
      [w8.reshape(R, D, D), root_w[None]], axis=0)

  table = _t1(node_features, w_all, root_b.reshape(1, D))

  hists = _sc_a1(dst, et)
  wt16 = _t2(hists)

  agg = _sc_a2(table, src, dst, et, wt16)
  out1 = _t3(table, agg[0], agg[1])

  neigh = _sc_b(out1, src, dst, et, wt16)
  out2 = _t4(neigh[0], neigh[1], out1, gc_w_rel, gc_w_root,
             gc_b.reshape(1, D))
  return out2


# speedup vs baseline: 1.5355x; 1.0200x over previous
import functools

import jax
import jax.numpy as jnp
from jax import lax
from jax.experimental import pallas as pl
from jax.experimental.pallas import tpu as pltpu
from jax.experimental.pallas import tpu_sc as plsc

N = 10000
E = 320000
D = 128
HD = D // 2
R = 8
NBASES = 30
RN = R * N

NC = 2
NS = 16
NW = NC * NS
L = 16

K = 128
CHUNKS = E // K

_GDN = lax.GatherDimensionNumbers(
    offset_dims=(), collapsed_slice_dims=(0,), start_index_map=(0,))


def _bcast_lane(vec, j):
  idx = jnp.full((L, 1), j, jnp.int32)
  return lax.gather(vec, idx, _GDN, slice_sizes=(1,),
                    mode=lax.GatherScatterMode.PROMISE_IN_BOUNDS)


def _num_chunks_for(wid):
  return (CHUNKS - wid + NW - 1) // NW


def _t0_body(comp_ref, basis_ref, out_ref):
  out_ref[...] = jnp.dot(comp_ref[...], basis_ref[...],
                         preferred_element_type=jnp.float32)


def _t0(comp, basis2d):
  g = basis2d.shape[1] // 2048
  return pl.pallas_call(
      _t0_body,
      grid=(g,),
      in_specs=[
          pl.BlockSpec((R, NBASES), lambda i: (0, 0)),
          pl.BlockSpec((NBASES, 2048), lambda i: (0, i)),
      ],
      out_specs=pl.BlockSpec((R, 2048), lambda i: (0, i)),
      out_shape=jax.ShapeDtypeStruct((R, basis2d.shape[1]), jnp.float32),
  )(comp, basis2d)


def _t1_body(x_ref, w_ref, b_ref, out_ref):
  o = jnp.dot(x_ref[...], w_ref[0], preferred_element_type=jnp.float32)
  bias = jnp.where(pl.program_id(1) == R, b_ref[...], 0.0)
  out_ref[...] = o + bias


def _t1(x, w_all, root_b2d):
  bn = 2000
  g = N // bn
  return pl.pallas_call(
      _t1_body,
      grid=(g, R + 1),
      in_specs=[
          pl.BlockSpec((bn, D), lambda i, r: (i, 0)),
          pl.BlockSpec((1, D, D), lambda i, r: (r, 0, 0)),
          pl.BlockSpec((1, D), lambda i, r: (0, 0)),
      ],
      out_specs=pl.BlockSpec((bn, D), lambda i, r: (r * g + i, 0)),
      out_shape=jax.ShapeDtypeStruct(((R + 1) * N, D), jnp.float32),
  )(x, w_all, root_b2d)


def _t2_body(h_ref, out_ref):
  s = jnp.sum(h_ref[...], axis=0, keepdims=True)
  out_ref[...] = 1.0 / jnp.maximum(s, 1.0)


def _t2(hists):
  bl = 3200
  g = RN // bl
  return pl.pallas_call(
      _t2_body,
      grid=(g,),
      in_specs=[pl.BlockSpec((NW, bl), lambda i: (0, i))],
      out_specs=pl.BlockSpec((1, bl), lambda i: (0, i)),
      out_shape=jax.ShapeDtypeStruct((1, RN), jnp.float32),
  )(hists)


def _t3_body(xr_ref, a0_ref, a1_ref, out_ref):
  out_ref[...] = xr_ref[...] + a0_ref[...] + a1_ref[...]


def _t3(table, a0, a1):
  bn = 2000
  g = N // bn
  spec = pl.BlockSpec((bn, D), lambda i: (i, 0))
  tspec = pl.BlockSpec((bn, D), lambda i: (R * g + i, 0))
  return pl.pallas_call(
      _t3_body,
      grid=(g,),
      in_specs=[tspec, spec, spec],
      out_specs=spec,
      out_shape=jax.ShapeDtypeStruct((N, D), jnp.float32),
  )(table, a0, a1)


def _t4_body(n0_ref, n1_ref, o1_ref, wr_ref, ww_ref, b_ref, out_ref):
  neigh = n0_ref[...] + n1_ref[...]
  f32 = jnp.float32
  out_ref[...] = (
      jnp.dot(neigh, wr_ref[...], preferred_element_type=f32)
      + jnp.dot(o1_ref[...], ww_ref[...], preferred_element_type=f32)
      + b_ref[...])


def _t4(n0, n1, o1, w_rel, w_root, b2d):
  bn = 2000
  g = N // bn
  spec = pl.BlockSpec((bn, D), lambda i: (i, 0))
  wspec = pl.BlockSpec((D, D), lambda i: (0, 0))
  return pl.pallas_call(
      _t4_body,
      grid=(g,),
      in_specs=[spec, spec, spec, wspec, wspec,
                pl.BlockSpec((1, D), lambda i: (0, 0))],
      out_specs=spec,
      out_shape=jax.ShapeDtypeStruct((N, D), jnp.float32),
  )(n0, n1, o1, w_rel, w_root, b2d)


_MESH = plsc.VectorSubcoreMesh(
    core_axis_name="c", subcore_axis_name="s", num_cores=NC, num_subcores=NS)
_SC_PARAMS = pltpu.CompilerParams(
    needs_layout_passes=False, use_tc_tiling_on_sc=False)


def _a1_body(dst_hbm, et_hbm, out_hbm, hist_v, dbuf, tbuf, dsem, tsem):
  wid = lax.axis_index("s") * NC + lax.axis_index("c")
  nk = _num_chunks_for(wid)
  lane = lax.iota(jnp.int32, L)
  zeros = jnp.zeros((L,), jnp.float32)
  ones = jnp.ones((L,), jnp.float32)

  def zero_body(i, _):
    hist_v[pl.ds(i * L, L)] = zeros
    return 0
  lax.fori_loop(0, RN // L, zero_body, 0)

  def fetch(k, b):
    @pl.when(k < nk)
    def _():
      c = wid + k * NW
      pltpu.async_copy(dst_hbm.at[pl.ds(c * K, K)], dbuf.at[b], dsem.at[b])
      pltpu.async_copy(et_hbm.at[pl.ds(c * K, K)], tbuf.at[b], tsem.at[b])

  def process(k, b):
    @pl.when(k < nk)
    def _():
      pltpu.make_async_copy(dst_hbm.at[pl.ds(0, K)], dbuf.at[b],
                            dsem.at[b]).wait()
      pltpu.make_async_copy(et_hbm.at[pl.ds(0, K)], tbuf.at[b],
                            tsem.at[b]).wait()

      def blk(i, _):
        dv = dbuf[b, pl.ds(i * L, L)]
        tv = tbuf[b, pl.ds(i * L, L)]
        civ = tv * N + dv
        for j in range(L):
          plsc.addupdate_scatter(hist_v, [civ], ones, mask=(lane == j))
        return 0
      lax.fori_loop(0, K // L, blk, 0)

  fetch(jnp.int32(0), 0)
  fetch(jnp.int32(1), 1)

  def body(kk, _):
    k = kk * 2
    process(k, 0)
    fetch(k + 2, 0)
    process(k + 1, 1)
    fetch(k + 3, 1)
    return 0
  lax.fori_loop(0, (nk + 1) // 2, body, 0)
  pltpu.sync_copy(hist_v, out_hbm.at[wid])


@functools.partial(
    pl.kernel,
    out_type=jax.ShapeDtypeStruct((NW, RN), jnp.float32),
    mesh=_MESH,
    scratch_types=[
        pltpu.VMEM((RN,), jnp.float32),
        pltpu.VMEM((2, K), jnp.int32),
        pltpu.VMEM((2, K), jnp.int32),
        pltpu.SemaphoreType.DMA((2,)),
        pltpu.SemaphoreType.DMA((2,)),
    ],
    compiler_params=_SC_PARAMS,
)
def _sc_a1(dst_hbm, et_hbm, out_hbm, hist_v, dbuf, tbuf, dsem, tsem):
  _a1_body(dst_hbm, et_hbm, out_hbm, hist_v, dbuf, tbuf, dsem, tsem)


def _sub_blocks(sid):
  start = sid * 624
  nblk = jnp.where(sid == NS - 1, 40, 39)
  return start, nblk


def _zero_spmem_slice(agg_sh, rows_v, sid):
  def zr(i, _):
    for h in range(rows_v.shape[-1] // L):
      rows_v[i, pl.ds(h * L, L)] = jnp.zeros((L,), jnp.float32)
    return 0
  lax.fori_loop(0, 16, zr, 0)
  start, nblk = _sub_blocks(sid)

  def cp(k, _):
    off = pl.multiple_of(start + k * 16, 8)
    pltpu.sync_copy(rows_v.at[pl.ds(0, 16)], agg_sh.at[pl.ds(off, 16)])
    return 0
  lax.fori_loop(0, nblk, cp, 0)


def _write_out_slice(agg_sh, out_hbm, sid):
  start, nblk = _sub_blocks(sid)

  def cp(k, _):
    off = pl.multiple_of(start + k * 16, 8)
    pltpu.sync_copy(agg_sh.at[pl.ds(off, 16)], out_hbm.at[pl.ds(off, 16)])
    return 0
  lax.fori_loop(0, nblk, cp, 0)


BINS_PER_W = RN // NW


def _w_body(wtab_hbm, out_hbm, wtab_v, wbuf):
  wid = lax.axis_index("s") * NC + lax.axis_index("c")
  pltpu.sync_copy(wtab_hbm, wtab_v)
  base = wid * BINS_PER_W

  def blk(i, _):
    wv = wtab_v[pl.ds(base + i * L, L)]
    for j in range(L):
      wbuf[i * L + j] = _bcast_lane(wv, j)
    return 0
  lax.fori_loop(0, BINS_PER_W // L, blk, 0)
  pltpu.sync_copy(wbuf, out_hbm.at[pl.ds(base, BINS_PER_W)])


@functools.partial(
    pl.kernel,
    out_type=jax.ShapeDtypeStruct((RN, L), jnp.float32),
    mesh=_MESH,
    scratch_types=[
        pltpu.VMEM((RN,), jnp.float32),
        pltpu.VMEM((BINS_PER_W, L), jnp.float32),
    ],
    compiler_params=_SC_PARAMS,
)
def _sc_w(wtab_hbm, out_hbm, wtab_v, wbuf):
  _w_body(wtab_hbm, out_hbm, wtab_v, wbuf)


def _agg_pipeline(table, src_hbm, dst_hbm, et_hbm, wt16_hbm, out_hbm,
                  agg, rows, sbuf, dbuf, tbuf, wbuf, gidx, widx, dstb,
                  ssem_i, dsem_i, tsem_i, wsem, gsem, ssem, scaled):
  cid = lax.axis_index("c")
  sid = lax.axis_index("s")
  wid = sid * NC + cid
  nk = _num_chunks_for(wid)

  _zero_spmem_slice(agg, rows.at[0], sid)
  plsc.subcore_barrier()

  def fetch(k, b):
    @pl.when(k < nk)
    def _():
      c = wid + k * NW
      pltpu.async_copy(src_hbm.at[pl.ds(c * K, K)], sbuf.at[b],
                       ssem_i.at[b])
      pltpu.async_copy(dst_hbm.at[pl.ds(c * K, K)], dbuf.at[b],
                       dsem_i.at[b])
      if scaled:
        pltpu.async_copy(et_hbm.at[pl.ds(c * K, K)], tbuf.at[b],
                         tsem_i.at[b])

  def gather(k, b):
    @pl.when(k < nk)
    def _():
      pltpu.make_async_copy(src_hbm.at[pl.ds(0, K)], sbuf.at[b],
                            ssem_i.at[b]).wait()
      if scaled:
        pltpu.make_async_copy(dst_hbm.at[pl.ds(0, K)], dbuf.at[b],
                              dsem_i.at[b]).wait()
        pltpu.make_async_copy(et_hbm.at[pl.ds(0, K)], tbuf.at[b],
                              tsem_i.at[b]).wait()

        def gix(i, _):
          sv = sbuf[b, pl.ds(i * L, L)]
          dv = dbuf[b, pl.ds(i * L, L)]
          tv = tbuf[b, pl.ds(i * L, L)]
          tvn = tv * N
          gidx[b, pl.ds(i * L, L)] = tvn + sv
          widx[b, pl.ds(i * L, L)] = tvn + dv
          return 0
        lax.fori_loop(0, K // L, gix, 0)
        src_idx = gidx.at[b]
        pltpu.async_copy(wt16_hbm.at[widx.at[b]], wbuf.at[b], wsem.at[b])
      else:
        src_idx = sbuf.at[b]

      @pl.when(k >= 2)
      def _():
        pltpu.make_async_copy(
            rows.at[b], agg.at[dstb.at[b]], ssem.at[b]).wait()
      pltpu.async_copy(table.at[src_idx], rows.at[b], gsem.at[b])

  def process(k, b):
    @pl.when(k < nk)
    def _():
      if scaled:
        src_idx = gidx.at[b]
        pltpu.make_async_copy(
            wt16_hbm.at[widx.at[b]], wbuf.at[b], wsem.at[b]).wait()
      else:
        src_idx = sbuf.at[b]
        pltpu.make_async_copy(dst_hbm.at[pl.ds(0, K)], dbuf.at[b],
                              dsem_i.at[b]).wait()
      pltpu.make_async_copy(table.at[src_idx], rows.at[b], gsem.at[b]).wait()

      def blk(i, _):
        dstb[b, pl.ds(i * L, L)] = dbuf[b, pl.ds(i * L, L)]
        if scaled:
          for j in range(L):
            e = i * L + j
            w = wbuf[b, e]
            for h in range(D // L):
              rows[b, e, pl.ds(h * L, L)] = rows[b, e, pl.ds(h * L, L)] * w
        return 0
      lax.fori_loop(0, K // L, blk, 0)

      pltpu.async_copy(rows.at[b], agg.at[dstb.at[b]], ssem.at[b],
                       add=True)

  fetch(jnp.int32(0), 0)
  fetch(jnp.int32(1), 1)
  gather(jnp.int32(0), 0)

  def body(kk, _):
    k = kk * 2
    process(k, 0)
    fetch(k + 2, 0)
    gather(k + 1, 1)
    process(k + 1, 1)
    fetch(k + 3, 1)
    gather(k + 2, 0)
    return 0
  lax.fori_loop(0, (nk + 1) // 2, body, 0)

  @pl.when(nk >= 2)
  def _():
    for b in range(2):
      pltpu.make_async_copy(
          rows.at[b], agg.at[dstb.at[b]], ssem.at[b]).wait()

  @pl.when(nk == 1)
  def _():
    pltpu.make_async_copy(
        rows.at[0], agg.at[dstb.at[0]], ssem.at[0]).wait()

  plsc.subcore_barrier()
  _write_out_slice(agg, out_hbm.at[cid], sid)


_AGG_SCRATCH = [
    pltpu.VMEM_SHARED((N, D), jnp.float32),
    pltpu.VMEM((2, K, D), jnp.float32),
    pltpu.VMEM((2, K), jnp.int32),
    pltpu.VMEM((2, K), jnp.int32),
    pltpu.VMEM((2, K), jnp.int32),
    pltpu.VMEM((2, K, L), jnp.float32),
    pltpu.VMEM((2, K), jnp.int32),
    pltpu.VMEM((2, K), jnp.int32),
    pltpu.VMEM((2, K), jnp.int32),
    pltpu.SemaphoreType.DMA((2,)),
    pltpu.SemaphoreType.DMA((2,)),
    pltpu.SemaphoreType.DMA((2,)),
    pltpu.SemaphoreType.DMA((2,)),
    pltpu.SemaphoreType.DMA((2,)),
    pltpu.SemaphoreType.DMA((2,)),
]


@functools.partial(
    pl.kernel,
    out_type=jax.ShapeDtypeStruct((NC, N, D), jnp.float32),
    mesh=_MESH,
    scratch_types=_AGG_SCRATCH,
    compiler_params=_SC_PARAMS,
)
def _sc_a2(table, src, dst, et, wt16, out, agg, rows,
           sbuf, dbuf, tbuf, wbuf, gidx, widx, dstb,
           ssem_i, dsem_i, tsem_i, wsem, gsem, ssem):
  _agg_pipeline(table, src, dst, et, wt16, out, agg, rows,
                sbuf, dbuf, tbuf, wbuf, gidx, widx, dstb,
                ssem_i, dsem_i, tsem_i, wsem, gsem, ssem, scaled=True)


@functools.partial(
    pl.kernel,
    out_type=jax.ShapeDtypeStruct((NC, N, D), jnp.float32),
    mesh=_MESH,
    scratch_types=_AGG_SCRATCH,
    compiler_params=_SC_PARAMS,
)
def _sc_b(table, src, dst, et, wt16, out, agg, rows,
          sbuf, dbuf, tbuf, wbuf, gidx, widx, dstb,
          ssem_i, dsem_i, tsem_i, wsem, gsem, ssem):
  _agg_pipeline(table, src, dst, et, wt16, out, agg, rows,
                sbuf, dbuf, tbuf, wbuf, gidx, widx, dstb,
                ssem_i, dsem_i, tsem_i, wsem, gsem, ssem, scaled=False)


def kernel(node_features, edge_index, edge_norm, edge_type, basis, comp,
           root_w, root_b, gc_w_rel, gc_w_root, gc_b):
  del edge_norm
  src = edge_index[0].astype(jnp.int32)
  dst = edge_index[1].astype(jnp.int32)
  et = edge_type.astype(jnp.int32)

  w8 = _t0(comp, basis.reshape(NBASES, D * D))
  w_all = jnp.concatenate(
      [w8.reshape(R, D, D), root_w[None]], axis=0)

  table = _t1(node_features, w_all, root_b.reshape(1, D))

  hists = _sc_a1(dst, et)
  wtab = _t2(hists).reshape(RN)
  wt16 = _sc_w(wtab)

  agg = _sc_a2(table, src, dst, et, wt16)
  out1 = _t3(table, agg[0], agg[1])

  neigh = _sc_b(out1, src, dst, et, wt16)
  out2 = _t4(neigh[0], neigh[1], out1, gc_w_rel, gc_w_root,
             gc_b.reshape(1, D))
  return out2
